# baseline (speedup 1.0000x reference)
import functools

import jax
import jax.numpy as jnp
from jax import lax
from jax.experimental import pallas as pl
from jax.experimental.pallas import tpu as pltpu
from jax.experimental.pallas import tpu_sc as plsc

VOCAB = 1000
BODY = 896
TAIL = VOCAB - BODY
NUM_WORKERS = 32
CHUNK = 40


def _gather_rows(body_hbm, tail_hbm, idx_hbm, out_hbm,
                 idx_v, buf0, buf1, tbuf0, tbuf1,
                 gb0, gb1, gt0, gt1, w0, w1):
    per_w = idx_v.shape[0]
    n_chunks = per_w // CHUNK
    wid = lax.axis_index("s") * 2 + lax.axis_index("c")
    base = wid * per_w
    pltpu.sync_copy(idx_hbm.at[pl.ds(base, per_w)], idx_v)

    bufs = (buf0, buf1)
    tbufs = (tbuf0, tbuf1)
    gbs = (gb0, gb1)
    gts = (gt0, gt1)
    ws = (w0, w1)

    lane = lax.iota(jnp.int32, 16)
    last_lanes = BODY + 6 * 16 + lane
    last_mask = lane < (TAIL - 6 * 16)
    last_idx = jnp.where(last_mask, last_lanes, VOCAB - 1)

    def start_gathers(g, slot):
        @pl.when(g >= 2)
        def _():
            wait_write(slot)
        off = pl.multiple_of(g * CHUNK, CHUNK)
        idx_c = idx_v.at[pl.ds(off, CHUNK)]
        pltpu.async_copy(body_hbm.at[idx_c],
                         bufs[slot].at[:, pl.ds(0, BODY)], gbs[slot])
        pltpu.async_copy(tail_hbm.at[idx_c], tbufs[slot], gts[slot])

    def wait_gathers(slot):
        pltpu.make_async_copy(body_hbm.at[pl.ds(0, CHUNK)],
                              bufs[slot].at[:, pl.ds(0, BODY)],
                              gbs[slot]).wait()
        pltpu.make_async_copy(tail_hbm.at[pl.ds(0, CHUNK)],
                              tbufs[slot], gts[slot]).wait()

    def wait_write(slot):
        pltpu.make_async_copy(bufs[slot], out_hbm.at[pl.ds(0, CHUNK)],
                              ws[slot]).wait()

    def splice(slot):
        buf, tbuf = bufs[slot], tbufs[slot]

        def splice_row(r, carry):
            for k in range(6):
                buf[r, pl.ds(BODY + 16 * k, 16)] = tbuf[r, pl.ds(16 * k, 16)]
            x = tbuf[r, pl.ds(96, 16)]
            plsc.store_scatter(buf, [jnp.full((16,), r, jnp.int32), last_idx],
                               x, mask=last_mask)
            return carry

        lax.fori_loop(0, CHUNK, splice_row, 0)

    def process(g, slot):
        wait_gathers(slot)
        splice(slot)
        off = pl.multiple_of(g * CHUNK, CHUNK)
        pltpu.async_copy(bufs[slot], out_hbm.at[pl.ds(base + off, CHUNK)],
                         ws[slot])

    start_gathers(0, 0)

    def ring(h, carry):
        g = 2 * h
        start_gathers(g + 1, 1)
        process(g, 0)

        @pl.when(g + 2 < n_chunks)
        def _():
            start_gathers(g + 2, 0)
        process(g + 1, 1)
        return carry

    lax.fori_loop(0, n_chunks // 2, ring, 0)
    wait_write(0)
    wait_write(1)


def kernel(idx, embed_weight):
    B, T = idx.shape
    N = B * T
    idx_flat = idx.reshape(N).astype(jnp.int32)
    body = embed_weight[:, :BODY]
    tail = jnp.pad(embed_weight[:, BODY:VOCAB], ((0, 0), (0, 128 - TAIL)))
    per_w = N // NUM_WORKERS

    mesh = plsc.VectorSubcoreMesh(core_axis_name="c", subcore_axis_name="s")
    k = functools.partial(
        pl.kernel,
        out_type=jax.ShapeDtypeStruct((N, VOCAB), jnp.float32),
        mesh=mesh,
        compiler_params=pltpu.CompilerParams(needs_layout_passes=False),
        scratch_types=[
            pltpu.VMEM((per_w,), jnp.int32),
            pltpu.VMEM((CHUNK, VOCAB), jnp.float32),
            pltpu.VMEM((CHUNK, VOCAB), jnp.float32),
            pltpu.VMEM((CHUNK, 128), jnp.float32),
            pltpu.VMEM((CHUNK, 128), jnp.float32),
            pltpu.SemaphoreType.DMA,
            pltpu.SemaphoreType.DMA,
            pltpu.SemaphoreType.DMA,
            pltpu.SemaphoreType.DMA,
            pltpu.SemaphoreType.DMA,
            pltpu.SemaphoreType.DMA,
        ],
    )(_gather_rows)
    out = k(body, tail, idx_flat)
    return out.reshape(B, T, VOCAB)
